# baseline (speedup 1.0000x reference)
import functools

import jax
import jax.numpy as jnp
from jax.experimental import pallas as pl
from jax.experimental.pallas import tpu as pltpu


def _load_gather_kernel(starts_ref, a_ref, s_ref, table_ref, out_ref,
                        scratch, sems, *, tile_b: int, n_slices: int,
                        slice_rows: int):
    i = pl.program_id(0)
    has_tail = table_ref.shape[0] % 8 != 0

    def _slice_copy(q):
        st = pl.multiple_of(starts_ref[q], 8)
        rows = slice_rows if q < n_slices else 8
        return pltpu.make_async_copy(
            table_ref.at[pl.ds(st, rows), :],
            scratch.at[pl.ds(st, rows), :],
            sems.at[q],
        )

    n_copies = n_slices + 1 if has_tail else n_slices

    @pl.when(i == 0)
    def _load():
        for q in range(n_copies):
            _slice_copy(q).start()
        for q in range(n_copies):
            _slice_copy(q).wait()

    base = i * tile_b
    for r in range(tile_b):
        a = pl.multiple_of(a_ref[base + r], 8)
        chunk = scratch[pl.ds(a, 8), :]
        rot = pltpu.roll(chunk, s_ref[base + r], axis=0)
        d = r & 7
        out_ref[pl.ds(r, 1), :] = rot[d:d + 1, :]


def kernel(labels, table, force_drop_ids):
    (B,) = labels.shape
    V, H = table.shape
    cfg_row = V - 1

    labels = labels.astype(jnp.int32)
    force_drop_ids = force_drop_ids.astype(jnp.int32)

    tile_b = 256
    while B % tile_b != 0:
        tile_b //= 2
    n_b = B // tile_b

    v_full = (V // 8) * 8
    n_slices = 8
    while v_full % (8 * n_slices) != 0:
        n_slices //= 2
    slice_rows = v_full // n_slices
    v_pad = ((V + 7) // 8) * 8
    starts = jnp.concatenate([
        jnp.arange(n_slices, dtype=jnp.int32) * slice_rows,
        jnp.array([v_full], dtype=jnp.int32),
    ])

    row = jnp.where(force_drop_ids == 1, cfg_row, labels)
    row = jnp.clip(row, 0, cfg_row)
    a = (row >> 3) << 3
    d = jnp.arange(B, dtype=jnp.int32) & 7
    shift = (d - (row - a)) & 7

    itemsize = jnp.dtype(table.dtype).itemsize

    grid_spec = pltpu.PrefetchScalarGridSpec(
        num_scalar_prefetch=3,
        grid=(n_b,),
        in_specs=[pl.BlockSpec(memory_space=pl.ANY)],
        out_specs=pl.BlockSpec((tile_b, H), lambda i, stv, av, sv: (i, 0)),
        scratch_shapes=[
            pltpu.VMEM((v_pad, H), table.dtype),
            pltpu.SemaphoreType.DMA((n_slices + 1,)),
        ],
    )
    out = pl.pallas_call(
        functools.partial(_load_gather_kernel, tile_b=tile_b,
                          n_slices=n_slices, slice_rows=slice_rows),
        out_shape=jax.ShapeDtypeStruct((B, H), table.dtype),
        grid_spec=grid_spec,
        compiler_params=pltpu.CompilerParams(
            dimension_semantics=("arbitrary",),
            vmem_limit_bytes=100 * 1024 * 1024,
            disable_bounds_checks=True,
        ),
        cost_estimate=pl.CostEstimate(
            flops=0,
            transcendentals=0,
            bytes_accessed=(V * H + B * H) * itemsize + 8 * B),
    )(starts, a, shift, table)
    return out


# speedup vs baseline: 1.0270x; 1.0270x over previous
import functools

import jax
import jax.numpy as jnp
from jax.experimental import pallas as pl
from jax.experimental.pallas import tpu as pltpu


def _vmem_gather_kernel(a_ref, s_ref, table_ref, out_ref,
                        *, tile_b: int):
    base = pl.program_id(0) * tile_b
    for r in range(tile_b):
        a = pl.multiple_of(a_ref[base + r], 8)
        chunk = table_ref[pl.ds(a, 8), :]
        rot = pltpu.roll(chunk, s_ref[base + r], axis=0)
        d = r & 7
        out_ref[pl.ds(r, 1), :] = rot[d:d + 1, :]


def kernel(labels, table, force_drop_ids):
    (B,) = labels.shape
    V, H = table.shape
    cfg_row = V - 1

    labels = labels.astype(jnp.int32)
    force_drop_ids = force_drop_ids.astype(jnp.int32)

    tile_b = 256
    while B % tile_b != 0:
        tile_b //= 2
    n_b = B // tile_b

    row = jnp.where(force_drop_ids == 1, cfg_row, labels)
    row = jnp.clip(row, 0, cfg_row)
    a = (row >> 3) << 3
    d = jnp.arange(B, dtype=jnp.int32) & 7
    shift = (d - (row - a)) & 7

    itemsize = jnp.dtype(table.dtype).itemsize

    grid_spec = pltpu.PrefetchScalarGridSpec(
        num_scalar_prefetch=2,
        grid=(n_b,),
        in_specs=[
            pl.BlockSpec((V, H), lambda i, av, sv: (0, 0),
                         pipeline_mode=pl.Buffered(1)),
        ],
        out_specs=pl.BlockSpec((tile_b, H), lambda i, av, sv: (i, 0)),
    )
    out = pl.pallas_call(
        functools.partial(_vmem_gather_kernel, tile_b=tile_b),
        out_shape=jax.ShapeDtypeStruct((B, H), table.dtype),
        grid_spec=grid_spec,
        compiler_params=pltpu.CompilerParams(
            dimension_semantics=("arbitrary",),
            vmem_limit_bytes=100 * 1024 * 1024,
            disable_bounds_checks=True,
        ),
        cost_estimate=pl.CostEstimate(
            flops=0,
            transcendentals=0,
            bytes_accessed=(V * H + B * H) * itemsize + 8 * B),
    )(a, shift, table)
    return out
